# baseline (speedup 1.0000x reference)
import functools

import jax
import jax.numpy as jnp
from jax.experimental import pallas as pl
from jax.experimental.pallas import tpu as pltpu

NUM_EXPERTS = 64
TOP_K = 8
BLOCK_T = 2048
BLOCK_K = 2048


def _router_block(x_ref, w_ref, ew_ref, idx_ref, logits_ref, probs_ref,
                  acc_ref, *, n_k):
    j = pl.program_id(1)

    @pl.when(j == 0)
    def _init():
        acc_ref[...] = jnp.zeros_like(acc_ref)

    acc_ref[...] += jnp.dot(x_ref[...], w_ref[...],
                            preferred_element_type=jnp.float32)

    @pl.when(j == n_k - 1)
    def _finish():
        logits = acc_ref[...]
        logits_ref[...] = logits

        iota_e = jax.lax.broadcasted_iota(jnp.int32, logits.shape, 1)
        work = logits
        mask = jnp.zeros(logits.shape, dtype=jnp.bool_)
        idx_cols = []
        for _ in range(TOP_K):
            m = jnp.max(work, axis=1, keepdims=True)
            cand = jnp.where(work == m, iota_e, NUM_EXPERTS)
            idx_k = jnp.min(cand, axis=1, keepdims=True)
            sel = iota_e == idx_k
            mask = jnp.logical_or(mask, sel)
            work = jnp.where(sel, -jnp.inf, work)
            idx_cols.append(idx_k)
        idx_ref[...] = jnp.concatenate(idx_cols, axis=1)

        m0 = jnp.max(logits, axis=1, keepdims=True)
        p = jnp.exp(logits - m0)
        probs_ref[...] = p / jnp.sum(p, axis=1, keepdims=True)
        p_sel = jnp.where(mask, p, 0.0)
        ew_ref[...] = p_sel / jnp.sum(p_sel, axis=1, keepdims=True)


@jax.jit
def kernel(inputs, gate_kernel):
    n_tokens, d_model = inputs.shape
    n_k = d_model // BLOCK_K
    grid = (n_tokens // BLOCK_T, n_k)
    out_shapes = (
        jax.ShapeDtypeStruct((n_tokens, NUM_EXPERTS), jnp.float32),
        jax.ShapeDtypeStruct((n_tokens, TOP_K), jnp.int32),
        jax.ShapeDtypeStruct((n_tokens, NUM_EXPERTS), jnp.float32),
        jax.ShapeDtypeStruct((n_tokens, NUM_EXPERTS), jnp.float32),
    )
    tok_spec = lambda w: pl.BlockSpec((BLOCK_T, w), lambda i, j: (i, 0))
    out = pl.pallas_call(
        functools.partial(_router_block, n_k=n_k),
        grid=grid,
        in_specs=[
            pl.BlockSpec((BLOCK_T, BLOCK_K), lambda i, j: (i, j)),
            pl.BlockSpec((BLOCK_K, NUM_EXPERTS), lambda i, j: (j, 0)),
        ],
        out_specs=(
            tok_spec(NUM_EXPERTS),
            tok_spec(TOP_K),
            tok_spec(NUM_EXPERTS),
            tok_spec(NUM_EXPERTS),
        ),
        out_shape=out_shapes,
        scratch_shapes=[pltpu.VMEM((BLOCK_T, NUM_EXPERTS), jnp.float32)],
        compiler_params=pltpu.CompilerParams(
            dimension_semantics=("arbitrary", "arbitrary"),
        ),
    )(inputs, gate_kernel)
    return out


# speedup vs baseline: 1.1267x; 1.1267x over previous
import functools

import jax
import jax.numpy as jnp
from jax import lax
from jax.experimental import pallas as pl
from jax.experimental.pallas import tpu as pltpu
from jax.experimental.pallas import tpu_sc as plsc

NUM_EXPERTS = 64
TOP_K = 8
BLOCK_T = 1024
TCH = 256
NC, NS = 2, 16
NW = NC * NS
CHW = NUM_EXPERTS * TCH


def _tc_block(x_ref, w_ref, logits_ref, probs_ref, lch_ref):
    logits = jnp.dot(x_ref[...], w_ref[...], preferred_element_type=jnp.float32)
    logits_ref[...] = logits
    m0 = jnp.max(logits, axis=1, keepdims=True)
    p = jnp.exp(logits - m0)
    probs_ref[...] = p / jnp.sum(p, axis=1, keepdims=True)
    nch = BLOCK_T // TCH
    lch_ref[...] = jnp.transpose(
        logits.reshape(nch, TCH, NUM_EXPERTS), (0, 2, 1))


def _sc_routing(lch_hbm, ew_hbm, idx_hbm, lbuf, ewbuf, idxbuf, *, ch_per_w):
    wid = lax.axis_index("s") * NC + lax.axis_index("c")
    lane = lax.iota(jnp.int32, 16)

    def chunk_body(c, carry):
        g = wid * ch_per_w + c
        pltpu.sync_copy(lch_hbm.at[pl.ds(g * CHW, CHW)], lbuf)

        def zbody(i, carry2):
            for u in range(8):
                ewbuf[pl.ds((i * 8 + u) * 16, 16)] = jnp.zeros((16,), jnp.float32)
            return carry2

        lax.fori_loop(0, TCH * NUM_EXPERTS // 128, zbody, 0)

        def gbody(lg, carry2):
            tok = lane + lg * 16
            vals, idxs = [], []
            for _ in range(TOP_K):
                parts = []
                for c4 in range(4):
                    be = c4 * 16
                    rm = lbuf[pl.ds(be * TCH + lg * 16, 16)]
                    ri = jnp.full((16,), be, jnp.int32)
                    for e in range(be + 1, be + 16):
                        v = lbuf[pl.ds(e * TCH + lg * 16, 16)]
                        take = v > rm
                        rm = jnp.where(take, v, rm)
                        ri = jnp.where(take, jnp.full((16,), e, jnp.int32), ri)
                    parts.append((rm, ri))
                rm, ri = parts[0]
                for v, vi in parts[1:]:
                    take = v > rm
                    rm = jnp.where(take, v, rm)
                    ri = jnp.where(take, vi, ri)
                vals.append(rm)
                idxs.append(ri)
                plsc.store_scatter(
                    lbuf, [ri * TCH + lg * 16 + lane],
                    jnp.full((16,), -1e30, jnp.float32))
            s = jnp.zeros((16,), jnp.float32)
            es = []
            for k in range(TOP_K):
                e_k = jnp.exp(vals[k] - vals[0])
                es.append(e_k)
                s = s + e_k
            rs = 1.0 / s
            for k in range(TOP_K):
                plsc.store_scatter(ewbuf, [tok * NUM_EXPERTS + idxs[k]], es[k] * rs)
                plsc.store_scatter(idxbuf, [tok * TOP_K + k], idxs[k])
            return carry2

        lax.fori_loop(0, TCH // 16, gbody, 0)
        tb = g * TCH
        pltpu.sync_copy(ewbuf, ew_hbm.at[pl.ds(tb * NUM_EXPERTS, TCH * NUM_EXPERTS)])
        pltpu.sync_copy(idxbuf, idx_hbm.at[pl.ds(tb * TOP_K, TCH * TOP_K)])
        return carry

    lax.fori_loop(0, ch_per_w, chunk_body, 0)


@jax.jit
def kernel(inputs, gate_kernel):
    n_tokens, d_model = inputs.shape
    grid = (n_tokens // BLOCK_T,)
    nch_blk = BLOCK_T // TCH
    n_chunks = n_tokens // TCH
    tok_spec = lambda w: pl.BlockSpec((BLOCK_T, w), lambda i: (i, 0))
    logits, probs, lchunks = pl.pallas_call(
        _tc_block,
        grid=grid,
        in_specs=[
            pl.BlockSpec((BLOCK_T, d_model), lambda i: (i, 0)),
            pl.BlockSpec((d_model, NUM_EXPERTS), lambda i: (0, 0)),
        ],
        out_specs=(
            tok_spec(NUM_EXPERTS),
            tok_spec(NUM_EXPERTS),
            pl.BlockSpec((nch_blk, NUM_EXPERTS, TCH), lambda i: (i, 0, 0)),
        ),
        out_shape=(
            jax.ShapeDtypeStruct((n_tokens, NUM_EXPERTS), jnp.float32),
            jax.ShapeDtypeStruct((n_tokens, NUM_EXPERTS), jnp.float32),
            jax.ShapeDtypeStruct((n_chunks, NUM_EXPERTS, TCH), jnp.float32),
        ),
        compiler_params=pltpu.CompilerParams(
            dimension_semantics=("arbitrary",),
        ),
    )(inputs, gate_kernel)

    ch_per_w = n_chunks // NW
    mesh = plsc.VectorSubcoreMesh(
        core_axis_name="c", subcore_axis_name="s",
        num_cores=NC, num_subcores=NS)
    ew_flat, idx_flat = pl.kernel(
        functools.partial(_sc_routing, ch_per_w=ch_per_w),
        out_type=(
            jax.ShapeDtypeStruct((n_tokens * NUM_EXPERTS,), jnp.float32),
            jax.ShapeDtypeStruct((n_tokens * TOP_K,), jnp.int32),
        ),
        mesh=mesh,
        compiler_params=pltpu.CompilerParams(needs_layout_passes=False),
        scratch_types=[
            pltpu.VMEM((CHW,), jnp.float32),
            pltpu.VMEM((TCH * NUM_EXPERTS,), jnp.float32),
            pltpu.VMEM((TCH * TOP_K,), jnp.int32),
        ],
    )(lchunks.reshape(-1))
    return (ew_flat.reshape(n_tokens, NUM_EXPERTS),
            idx_flat.reshape(n_tokens, TOP_K),
            logits, probs)


# speedup vs baseline: 1.2867x; 1.1420x over previous
import functools

import jax
import jax.numpy as jnp
from jax import lax
from jax.experimental import pallas as pl
from jax.experimental.pallas import tpu as pltpu
from jax.experimental.pallas import tpu_sc as plsc

NUM_EXPERTS = 64
TOP_K = 8
BLOCK_T = 1024
TCH = 256
NC, NS = 2, 16
NW = NC * NS


def _tc_block(x_ref, w_ref, logits_ref, probs_ref, lch_ref):
    logits = jnp.dot(x_ref[...], w_ref[...], preferred_element_type=jnp.float32)
    logits_ref[...] = logits
    m0 = jnp.max(logits, axis=1, keepdims=True)
    p = jnp.exp(logits - m0)
    probs_ref[...] = p / jnp.sum(p, axis=1, keepdims=True)
    nch = BLOCK_T // TCH
    lch_ref[...] = jnp.transpose(
        logits.reshape(nch, TCH, NUM_EXPERTS), (0, 2, 1))


def _sc_routing(lch_hbm, ew_hbm, idx_hbm, lbuf, ewbuf, idxbuf, *, ch_per_w):
    wid = lax.axis_index("s") * NC + lax.axis_index("c")
    lane = lax.iota(jnp.int32, 16)

    def chunk_body(c, carry):
        g = wid * ch_per_w + c
        pltpu.sync_copy(lch_hbm.at[g], lbuf)

        def zbody(i, carry2):
            for u in range(NUM_EXPERTS // 16):
                ewbuf[i, pl.ds(u * 16, 16)] = jnp.zeros((16,), jnp.float32)
            return carry2

        lax.fori_loop(0, TCH, zbody, 0)

        def gbody(lg, carry2):
            tok = lane + lg * 16
            vals, idxs = [], []
            for _ in range(TOP_K):
                parts = []
                for c4 in range(4):
                    be = c4 * 16
                    rm = lbuf[be, pl.ds(lg * 16, 16)]
                    ri = jnp.full((16,), be, jnp.int32)
                    for e in range(be + 1, be + 16):
                        v = lbuf[e, pl.ds(lg * 16, 16)]
                        take = v > rm
                        rm = jnp.where(take, v, rm)
                        ri = jnp.where(take, jnp.full((16,), e, jnp.int32), ri)
                    parts.append((rm, ri))
                rm, ri = parts[0]
                for v, vi in parts[1:]:
                    take = v > rm
                    rm = jnp.where(take, v, rm)
                    ri = jnp.where(take, vi, ri)
                vals.append(rm)
                idxs.append(ri)
                plsc.store_scatter(
                    lbuf, [ri, lg * 16 + lane],
                    jnp.full((16,), -1e30, jnp.float32))
            s = jnp.zeros((16,), jnp.float32)
            es = []
            for k in range(TOP_K):
                e_k = jnp.exp(vals[k] - vals[0])
                es.append(e_k)
                s = s + e_k
            rs = 1.0 / s
            for k in range(TOP_K):
                plsc.store_scatter(ewbuf, [tok, idxs[k]], es[k] * rs)
                plsc.store_scatter(idxbuf, [tok, jnp.full((16,), k, jnp.int32)],
                                   idxs[k])
            return carry2

        lax.fori_loop(0, TCH // 16, gbody, 0)
        tb = g * TCH
        pltpu.sync_copy(ewbuf, ew_hbm.at[pl.ds(tb, TCH), :])
        pltpu.sync_copy(idxbuf, idx_hbm.at[pl.ds(tb, TCH), :])
        return carry

    lax.fori_loop(0, ch_per_w, chunk_body, 0)


@jax.jit
def kernel(inputs, gate_kernel):
    n_tokens, d_model = inputs.shape
    grid = (n_tokens // BLOCK_T,)
    nch_blk = BLOCK_T // TCH
    n_chunks = n_tokens // TCH
    tok_spec = lambda w: pl.BlockSpec((BLOCK_T, w), lambda i: (i, 0))
    logits, probs, lchunks = pl.pallas_call(
        _tc_block,
        grid=grid,
        in_specs=[
            pl.BlockSpec((BLOCK_T, d_model), lambda i: (i, 0)),
            pl.BlockSpec((d_model, NUM_EXPERTS), lambda i: (0, 0)),
        ],
        out_specs=(
            tok_spec(NUM_EXPERTS),
            tok_spec(NUM_EXPERTS),
            pl.BlockSpec((nch_blk, NUM_EXPERTS, TCH), lambda i: (i, 0, 0)),
        ),
        out_shape=(
            jax.ShapeDtypeStruct((n_tokens, NUM_EXPERTS), jnp.float32),
            jax.ShapeDtypeStruct((n_tokens, NUM_EXPERTS), jnp.float32),
            jax.ShapeDtypeStruct((n_chunks, NUM_EXPERTS, TCH), jnp.float32),
        ),
        compiler_params=pltpu.CompilerParams(
            dimension_semantics=("arbitrary",),
        ),
    )(inputs, gate_kernel)

    ch_per_w = n_chunks // NW
    mesh = plsc.VectorSubcoreMesh(
        core_axis_name="c", subcore_axis_name="s",
        num_cores=NC, num_subcores=NS)
    ew, idx = pl.kernel(
        functools.partial(_sc_routing, ch_per_w=ch_per_w),
        out_type=(
            jax.ShapeDtypeStruct((n_tokens, NUM_EXPERTS), jnp.float32),
            jax.ShapeDtypeStruct((n_tokens, TOP_K), jnp.int32),
        ),
        mesh=mesh,
        compiler_params=pltpu.CompilerParams(needs_layout_passes=False),
        scratch_types=[
            pltpu.VMEM((NUM_EXPERTS, TCH), jnp.float32),
            pltpu.VMEM((TCH, NUM_EXPERTS), jnp.float32),
            pltpu.VMEM((TCH, TOP_K), jnp.int32),
        ],
    )(lchunks)
    return (ew, idx, logits, probs)
